# Initial kernel scaffold; baseline (speedup 1.0000x reference)
import jax
import jax.numpy as jnp
from jax.experimental import pallas as pl


def kernel(x_gene, x_disease, edge_index_g2d, edge_index_d2g, edge_index_g2g, proj_gene_w, proj_gene_b, proj_dis_w, proj_dis_b, a_src_g2d, a_dst_g2d, a_src_d2g, a_dst_d2g, a_src_g2g, a_dst_g2g, k_w, k_b, q, lin_w, lin_b):
    raise NotImplementedError("write your pallas kernel here")


# speedup vs baseline: 11.5683x; 11.5683x over previous
import functools

import jax
import jax.numpy as jnp
from jax import lax
from jax.experimental import pallas as pl
from jax.experimental.pallas import tpu as pltpu
from jax.experimental.pallas import tpu_sc as plsc

N = 50000
D_IN = 128
HID = 128
HEADS = 8
D_HEAD = 16
OUT = 64
E = 200000

K = 256
NCH = (E + K - 1) // K
E_PAD = NCH * K
R = 10240
NPASS = 5
NOUT = R * NPASS
CEMIT = 128
RPT = R // 16
DPT = NOUT // 16


def _iota16():
    return lax.broadcasted_iota(jnp.int32, (16,), 0)


def _lrelu_exp(a):
    x = jnp.where(a >= 0.0, a, 0.2 * a)
    return jnp.exp(x)


def _sc_process(xsrc, atab, src_h, dst_h, zacc_h, zden_h,
                acc_o, den_o,
                den_sp, acc_sp, srcv, dstv,
                pend_s, pend_d, pend_v, ov_s, ov_d, ov_v,
                rows, arows, brows, exbuf, dl,
                sem_a, sem_b, sem_c, sid):
    it16 = _iota16()
    nmy = (NCH - sid + 15) // 16

    pltpu.sync_copy(zden_h, den_sp.at[pl.ds(sid * DPT, DPT)])

    r0 = sid * RPT

    def p2_pass(p, carry):
        base = p * R

        pltpu.sync_copy(zacc_h, acc_sp.at[pl.ds(r0, RPT)])
        plsc.subcore_barrier()

        def emit(fill_after):
            dx = pltpu.async_copy(xsrc.at[pend_s], rows, sem_a)
            da = pltpu.async_copy(atab.at[pend_s], arows, sem_b)
            db = pltpu.async_copy(atab.at[pend_d], brows, sem_c)
            dx.wait()
            da.wait()
            db.wait()

            def exj(j, c2):
                e16 = j * 16 + it16
                v = pend_v[pl.ds(j * 16, 16)]
                for h in range(HEADS):
                    h16 = jnp.full((16,), h, jnp.int32)
                    a = plsc.load_gather(arows, [e16, h16])
                    b = plsc.load_gather(brows, [e16, h16 + 8])
                    ex = _lrelu_exp(a + b) * v
                    plsc.store_scatter(exbuf, [e16, h16], ex)
                return c2

            lax.fori_loop(0, CEMIT // 16, exj, 0)
            pltpu.sync_copy(exbuf, den_sp.at[pend_d], add=True)

            for j in range(CEMIT // 16):
                d16 = pend_d[pl.ds(j * 16, 16)]
                dl[pl.ds(j * 16, 16)] = d16 - base

            zi16 = jnp.zeros((16,), jnp.int32)

            def scale(e, c2):
                for h in range(HEADS):
                    sv = plsc.load_gather(
                        exbuf, [zi16 + e, jnp.full((16,), h, jnp.int32)])
                    rows[e, pl.ds(h * 16, 16)] = (
                        rows[e, pl.ds(h * 16, 16)] * sv)
                return c2

            lax.fori_loop(0, CEMIT, scale, 0)
            pltpu.sync_copy(rows, acc_sp.at[dl], add=True)
            return fill_after

        def p2_chunk(gl, fill):
            g = sid + gl * 16
            base_e = g * K
            pltpu.sync_copy(src_h.at[pl.ds(base_e, K)], srcv)
            pltpu.sync_copy(dst_h.at[pl.ds(base_e, K)], dstv)

            def p2_vec(i, fill):
                e16 = i * 16 + it16
                eid = base_e + e16
                s16 = srcv[pl.ds(i * 16, 16)]
                d16 = dstv[pl.ds(i * 16, 16)]
                m = (d16 >= base) & (d16 < base + R) & (eid < E)
                cnt = jnp.sum(m.astype(jnp.int32))
                pos = fill + plsc.cumsum(m.astype(jnp.int32)) - 1
                m_lo = m & (pos < CEMIT)
                m_hi = m & (pos >= CEMIT)
                one16 = jnp.full((16,), 1.0, jnp.float32)
                plsc.store_scatter(pend_s, [pos], s16, mask=m_lo)
                plsc.store_scatter(pend_d, [pos], d16, mask=m_lo)
                plsc.store_scatter(pend_v, [pos], one16, mask=m_lo)
                plsc.store_scatter(ov_s, [pos - CEMIT], s16, mask=m_hi)
                plsc.store_scatter(ov_d, [pos - CEMIT], d16, mask=m_hi)
                plsc.store_scatter(ov_v, [pos - CEMIT], one16, mask=m_hi)
                fill = fill + cnt

                def do_emit(f):
                    f = emit(f - CEMIT)
                    pend_s[pl.ds(0, 16)] = ov_s[...]
                    pend_d[pl.ds(0, 16)] = ov_d[...]
                    pend_v[pl.ds(0, 16)] = ov_v[...]
                    return f

                return lax.cond(fill >= CEMIT, do_emit, lambda f: f, fill)

            return lax.fori_loop(0, K // 16, p2_vec, fill)

        fill = lax.fori_loop(0, nmy, p2_chunk, jnp.int32(0))

        dummy_s = sid * 16 + it16
        zero16 = jnp.zeros((16,), jnp.float32)
        for jj in range(CEMIT // 16):
            pos = fill + jj * 16 + it16
            mk = pos < CEMIT
            dummy_d = base + sid * CEMIT + jj * 16 + it16
            plsc.store_scatter(pend_s, [pos], dummy_s, mask=mk)
            plsc.store_scatter(pend_d, [pos], dummy_d, mask=mk)
            plsc.store_scatter(pend_v, [pos], zero16, mask=mk)
        emit(0)

        plsc.subcore_barrier()
        pltpu.sync_copy(acc_sp.at[pl.ds(r0, RPT)],
                        acc_o.at[pl.ds(base + r0, RPT)])
        plsc.subcore_barrier()
        return carry

    lax.fori_loop(0, NPASS, p2_pass, 0)
    pltpu.sync_copy(den_sp.at[pl.ds(sid * DPT, DPT)],
                    den_o.at[pl.ds(sid * DPT, DPT)])


def _sc_body(hd, hg, t1, t2, s1, d1, s2, d2, zacc_h, zden_h,
             acc1, den1, acc2, den2,
             den_sp, acc_sp, srcv, dstv,
             pend_s, pend_d, pend_v, ov_s, ov_d, ov_v,
             rows, arows, brows, exbuf, dl,
             sem_a, sem_b, sem_c):
    cid = lax.axis_index("c")
    sid = lax.axis_index("s")

    args = (den_sp, acc_sp, srcv, dstv,
            pend_s, pend_d, pend_v, ov_s, ov_d, ov_v,
            rows, arows, brows, exbuf, dl,
            sem_a, sem_b, sem_c, sid)

    @pl.when(cid == 0)
    def _():
        _sc_process(hd, t1, s1, d1, zacc_h, zden_h, acc1, den1, *args)

    @pl.when(cid == 1)
    def _():
        _sc_process(hg, t2, s2, d2, zacc_h, zden_h, acc2, den2, *args)


def _sc_edge_kernel(hd, hg, t1, t2, s1, d1, s2, d2, zacc_h, zden_h):
    f32 = jnp.float32
    i32 = jnp.int32
    mesh = plsc.VectorSubcoreMesh(core_axis_name="c", subcore_axis_name="s")
    return pl.kernel(
        _sc_body,
        out_type=(
            jax.ShapeDtypeStruct((NOUT, HID), f32),
            jax.ShapeDtypeStruct((NOUT, HEADS), f32),
            jax.ShapeDtypeStruct((NOUT, HID), f32),
            jax.ShapeDtypeStruct((NOUT, HEADS), f32),
        ),
        mesh=mesh,
        compiler_params=pltpu.CompilerParams(use_tc_tiling_on_sc=False,
                                             needs_layout_passes=False),
        scratch_types=[
            pltpu.VMEM_SHARED((NOUT, HEADS), f32),
            pltpu.VMEM_SHARED((R, HID), f32),
            pltpu.VMEM((K,), i32),
            pltpu.VMEM((K,), i32),
            pltpu.VMEM((CEMIT,), i32),
            pltpu.VMEM((CEMIT,), i32),
            pltpu.VMEM((CEMIT,), f32),
            pltpu.VMEM((16,), i32),
            pltpu.VMEM((16,), i32),
            pltpu.VMEM((16,), f32),
            pltpu.VMEM((CEMIT, HID), f32),
            pltpu.VMEM((CEMIT, 2 * HEADS), f32),
            pltpu.VMEM((CEMIT, 2 * HEADS), f32),
            pltpu.VMEM((CEMIT, HEADS), f32),
            pltpu.VMEM((CEMIT,), i32),
            pltpu.SemaphoreType.DMA,
            pltpu.SemaphoreType.DMA,
            pltpu.SemaphoreType.DMA,
        ],
    )(hd, hg, t1, t2, s1, d1, s2, d2, zacc_h, zden_h)


BM = 400
GRID = N // BM


def _proj_body(xg, xd, wg, bg, wd, bd, m1, m2, m34,
               hg, hd, t1, t2):
    g = jnp.dot(xg[...], wg[...], preferred_element_type=jnp.float32) + bg[...]
    d = jnp.dot(xd[...], wd[...], preferred_element_type=jnp.float32) + bd[...]
    hg[...] = g
    hd[...] = d
    t1[...] = (jnp.dot(d, m1[...], preferred_element_type=jnp.float32)
               + jnp.dot(g, m2[...], preferred_element_type=jnp.float32))
    t2[...] = jnp.dot(g, m34[...], preferred_element_type=jnp.float32)


def _proj(xg, xd, wg, bg, wd, bd, m1, m2, m34):
    f32 = jnp.float32
    row = lambda i: (i, 0)
    whole = lambda i: (0, 0)
    return pl.pallas_call(
        _proj_body,
        grid=(GRID,),
        in_specs=[
            pl.BlockSpec((BM, D_IN), row),
            pl.BlockSpec((BM, D_IN), row),
            pl.BlockSpec((D_IN, HID), whole),
            pl.BlockSpec((1, HID), whole),
            pl.BlockSpec((D_IN, HID), whole),
            pl.BlockSpec((1, HID), whole),
            pl.BlockSpec((HID, 2 * HEADS), whole),
            pl.BlockSpec((HID, 2 * HEADS), whole),
            pl.BlockSpec((HID, 2 * HEADS), whole),
        ],
        out_specs=[
            pl.BlockSpec((BM, HID), row),
            pl.BlockSpec((BM, HID), row),
            pl.BlockSpec((BM, 2 * HEADS), row),
            pl.BlockSpec((BM, 2 * HEADS), row),
        ],
        out_shape=[
            jax.ShapeDtypeStruct((N, HID), f32),
            jax.ShapeDtypeStruct((N, HID), f32),
            jax.ShapeDtypeStruct((N, 2 * HEADS), f32),
            jax.ShapeDtypeStruct((N, 2 * HEADS), f32),
        ],
    )(xg, xd, wg, bg, wd, bd, m1, m2, m34)


def _normalize(acc, den, expand):
    r = 1.0 / (den[...] + 1e-16)
    rx = jnp.dot(r, expand[...], preferred_element_type=jnp.float32)
    return jnp.maximum(acc[...] * rx, 0.0)


def _c1_body(acc1, den1, acc2, den2, kw, kb, expand, ks1, ks2):
    i = pl.program_id(0)

    @pl.when(i == 0)
    def _():
        ks1[...] = jnp.zeros_like(ks1)
        ks2[...] = jnp.zeros_like(ks2)

    o1 = _normalize(acc1, den1, expand)
    o2 = _normalize(acc2, den2, expand)
    k1 = jnp.tanh(jnp.dot(o1, kw[...], preferred_element_type=jnp.float32)
                  + kb[...])
    k2 = jnp.tanh(jnp.dot(o2, kw[...], preferred_element_type=jnp.float32)
                  + kb[...])
    ks1[...] += jnp.sum(k1, axis=0, keepdims=True)
    ks2[...] += jnp.sum(k2, axis=0, keepdims=True)


def _c1(acc1, den1, acc2, den2, kw, kb, expand):
    f32 = jnp.float32
    row = lambda i: (i, 0)
    whole = lambda i: (0, 0)
    return pl.pallas_call(
        _c1_body,
        grid=(GRID,),
        in_specs=[
            pl.BlockSpec((BM, HID), row),
            pl.BlockSpec((BM, HEADS), row),
            pl.BlockSpec((BM, HID), row),
            pl.BlockSpec((BM, HEADS), row),
            pl.BlockSpec((HID, HID), whole),
            pl.BlockSpec((1, HID), whole),
            pl.BlockSpec((HEADS, HID), whole),
        ],
        out_specs=[
            pl.BlockSpec((1, HID), whole),
            pl.BlockSpec((1, HID), whole),
        ],
        out_shape=[
            jax.ShapeDtypeStruct((1, HID), f32),
            jax.ShapeDtypeStruct((1, HID), f32),
        ],
    )(acc1, den1, acc2, den2, kw, kb, expand)


def _c2_body(acc1, den1, acc2, den2, attn, lw, lb, expand, out):
    o1 = _normalize(acc1, den1, expand)
    o2 = _normalize(acc2, den2, expand)
    a1 = attn[0, 0]
    a2 = attn[0, 1]
    o = a1 * o1 + a2 * o2
    out[...] = jnp.dot(o, lw[...], preferred_element_type=jnp.float32) + lb[...]


def _c2(acc1, den1, acc2, den2, attn, lw, lb, expand):
    f32 = jnp.float32
    row = lambda i: (i, 0)
    whole = lambda i: (0, 0)
    return pl.pallas_call(
        _c2_body,
        grid=(GRID,),
        in_specs=[
            pl.BlockSpec((BM, HID), row),
            pl.BlockSpec((BM, HEADS), row),
            pl.BlockSpec((BM, HID), row),
            pl.BlockSpec((BM, HEADS), row),
            pl.BlockSpec(memory_space=pltpu.SMEM),
            pl.BlockSpec((HID, OUT), whole),
            pl.BlockSpec((1, OUT), whole),
            pl.BlockSpec((HEADS, HID), whole),
        ],
        out_specs=pl.BlockSpec((BM, OUT), row),
        out_shape=jax.ShapeDtypeStruct((N, OUT), f32),
    )(acc1, den1, acc2, den2, attn, lw, lb, expand)


def _head_mat(a):
    return (a[:, :, None] * jnp.eye(HEADS, dtype=a.dtype)[:, None, :]).reshape(
        HID, HEADS)


def _pad_edges(e):
    return jnp.pad(e, (0, E_PAD - E))


def kernel(x_gene, x_disease, edge_index_g2d, edge_index_d2g, edge_index_g2g,
           proj_gene_w, proj_gene_b, proj_dis_w, proj_dis_b,
           a_src_g2d, a_dst_g2d, a_src_d2g, a_dst_d2g, a_src_g2g, a_dst_g2g,
           k_w, k_b, q, lin_w, lin_b):
    zpad = jnp.zeros((HID, HEADS), jnp.float32)
    m1 = jnp.concatenate([_head_mat(a_src_d2g), zpad], axis=1)
    m2 = jnp.concatenate([zpad, _head_mat(a_dst_d2g)], axis=1)
    m34 = jnp.concatenate([_head_mat(a_src_g2g), _head_mat(a_dst_g2g)],
                          axis=1)
    hg, hd, t1, t2 = _proj(
        x_gene, x_disease, proj_gene_w, proj_gene_b.reshape(1, HID),
        proj_dis_w, proj_dis_b.reshape(1, HID), m1, m2, m34)
    expand = (jnp.eye(HEADS, dtype=jnp.float32)[:, :, None]
              * jnp.ones((1, 1, D_HEAD), jnp.float32)).reshape(HEADS, HID)

    s1 = _pad_edges(edge_index_d2g[0])
    d1 = _pad_edges(edge_index_d2g[1])
    s2 = _pad_edges(edge_index_g2g[0])
    d2 = _pad_edges(edge_index_g2g[1])

    zacc_h = jnp.zeros((RPT, HID), jnp.float32)
    zden_h = jnp.zeros((DPT, HEADS), jnp.float32)
    acc1, den1, acc2, den2 = _sc_edge_kernel(
        hd, hg, t1, t2, s1, d1, s2, d2, zacc_h, zden_h)

    ks1, ks2 = _c1(acc1, den1, acc2, den2, k_w, k_b.reshape(1, HID),
                   expand)
    s_1 = jnp.dot(q, ks1[0] / N)
    s_2 = jnp.dot(q, ks2[0] / N)
    attn = jax.nn.softmax(jnp.stack([s_1, s_2])).reshape(1, 2)

    return _c2(acc1, den1, acc2, den2, attn, lin_w,
               lin_b.reshape(1, OUT), expand)


# speedup vs baseline: 14.7868x; 1.2782x over previous
import functools

import jax
import jax.numpy as jnp
from jax import lax
from jax.experimental import pallas as pl
from jax.experimental.pallas import tpu as pltpu
from jax.experimental.pallas import tpu_sc as plsc

N = 50000
D_IN = 128
HID = 128
HEADS = 8
D_HEAD = 16
OUT = 64
E = 200000

K = 256
NCH = (E + K - 1) // K
E_PAD = NCH * K
R = 10240
NPASS = 5
NOUT = R * NPASS
CEMIT = 128
RPT = R // 16
DPT = NOUT // 16


def _iota16():
    return lax.broadcasted_iota(jnp.int32, (16,), 0)


def _lrelu_exp(a):
    x = jnp.where(a >= 0.0, a, 0.2 * a)
    return jnp.exp(x)


def _sc_process(xsrc, atab, src_h, dst_h, zacc_h, zden_h,
                acc_o, den_o,
                den_sp, acc_sp, srcv, dstv,
                pend_s, pend_d, pend_v, ov_s, ov_d, ov_v,
                rows, arows, brows, exbuf, dl,
                sem_a, sem_b, sem_c, sid):
    it16 = _iota16()
    nmy = (NCH - sid + 15) // 16

    pltpu.sync_copy(zden_h, den_sp.at[pl.ds(sid * DPT, DPT)])

    r0 = sid * RPT

    def p2_pass(p, carry):
        base = p * R

        pltpu.sync_copy(zacc_h, acc_sp.at[pl.ds(r0, RPT)])
        plsc.subcore_barrier()

        def emit(fill_after):
            dx = pltpu.async_copy(xsrc.at[pend_s], rows, sem_a)
            da = pltpu.async_copy(atab.at[pend_s], arows, sem_b)
            db = pltpu.async_copy(atab.at[pend_d], brows, sem_c)
            dx.wait()
            da.wait()
            db.wait()

            def exj(j, c2):
                e16 = j * 16 + it16
                v = pend_v[pl.ds(j * 16, 16)]
                for h in range(HEADS):
                    h16 = jnp.full((16,), h, jnp.int32)
                    a = plsc.load_gather(arows, [e16, h16])
                    b = plsc.load_gather(brows, [e16, h16 + 8])
                    ex = _lrelu_exp(a + b) * v
                    plsc.store_scatter(exbuf, [e16, h16], ex)
                    plsc.store_scatter(brows, [e16, h16], ex)
                return c2

            lax.fori_loop(0, CEMIT // 16, exj, 0)
            pltpu.sync_copy(exbuf, den_sp.at[pend_d], add=True)

            for j in range(CEMIT // 16):
                d16 = pend_d[pl.ds(j * 16, 16)]
                dl[pl.ds(j * 16, 16)] = d16 - base

            gdn = lax.GatherDimensionNumbers(
                offset_dims=(), collapsed_slice_dims=(0,),
                start_index_map=(0,))

            def scale(e, c2):
                ex16 = brows[e, pl.ds(0, 16)]
                for h in range(HEADS):
                    sv = lax.gather(
                        ex16, jnp.full((16, 1), h, jnp.int32), gdn,
                        slice_sizes=(1,),
                        mode=lax.GatherScatterMode.PROMISE_IN_BOUNDS)
                    rows[e, pl.ds(h * 16, 16)] = (
                        rows[e, pl.ds(h * 16, 16)] * sv)
                return c2

            lax.fori_loop(0, CEMIT, scale, 0)
            pltpu.sync_copy(rows, acc_sp.at[dl], add=True)
            return fill_after

        def p2_chunk(gl, fill):
            g = sid + gl * 16
            base_e = g * K
            pltpu.sync_copy(src_h.at[pl.ds(base_e, K)], srcv)
            pltpu.sync_copy(dst_h.at[pl.ds(base_e, K)], dstv)

            def p2_vec(i, fill):
                e16 = i * 16 + it16
                eid = base_e + e16
                s16 = srcv[pl.ds(i * 16, 16)]
                d16 = dstv[pl.ds(i * 16, 16)]
                m = (d16 >= base) & (d16 < base + R) & (eid < E)
                cnt = jnp.sum(m.astype(jnp.int32))
                pos = fill + plsc.cumsum(m.astype(jnp.int32)) - 1
                m_lo = m & (pos < CEMIT)
                m_hi = m & (pos >= CEMIT)
                one16 = jnp.full((16,), 1.0, jnp.float32)
                plsc.store_scatter(pend_s, [pos], s16, mask=m_lo)
                plsc.store_scatter(pend_d, [pos], d16, mask=m_lo)
                plsc.store_scatter(pend_v, [pos], one16, mask=m_lo)
                plsc.store_scatter(ov_s, [pos - CEMIT], s16, mask=m_hi)
                plsc.store_scatter(ov_d, [pos - CEMIT], d16, mask=m_hi)
                plsc.store_scatter(ov_v, [pos - CEMIT], one16, mask=m_hi)
                fill = fill + cnt

                def do_emit(f):
                    f = emit(f - CEMIT)
                    pend_s[pl.ds(0, 16)] = ov_s[...]
                    pend_d[pl.ds(0, 16)] = ov_d[...]
                    pend_v[pl.ds(0, 16)] = ov_v[...]
                    return f

                return lax.cond(fill >= CEMIT, do_emit, lambda f: f, fill)

            return lax.fori_loop(0, K // 16, p2_vec, fill)

        fill = lax.fori_loop(0, nmy, p2_chunk, jnp.int32(0))

        dummy_s = sid * 16 + it16
        zero16 = jnp.zeros((16,), jnp.float32)
        for jj in range(CEMIT // 16):
            pos = fill + jj * 16 + it16
            mk = pos < CEMIT
            dummy_d = base + sid * CEMIT + jj * 16 + it16
            plsc.store_scatter(pend_s, [pos], dummy_s, mask=mk)
            plsc.store_scatter(pend_d, [pos], dummy_d, mask=mk)
            plsc.store_scatter(pend_v, [pos], zero16, mask=mk)
        emit(0)

        plsc.subcore_barrier()
        pltpu.sync_copy(acc_sp.at[pl.ds(r0, RPT)],
                        acc_o.at[pl.ds(base + r0, RPT)])
        plsc.subcore_barrier()
        return carry

    lax.fori_loop(0, NPASS, p2_pass, 0)
    pltpu.sync_copy(den_sp.at[pl.ds(sid * DPT, DPT)],
                    den_o.at[pl.ds(sid * DPT, DPT)])


def _sc_body(hd, hg, t1, t2, s1, d1, s2, d2, zacc_h, zden_h,
             acc1, den1, acc2, den2,
             den_sp, acc_sp, srcv, dstv,
             pend_s, pend_d, pend_v, ov_s, ov_d, ov_v,
             rows, arows, brows, exbuf, dl,
             sem_a, sem_b, sem_c):
    cid = lax.axis_index("c")
    sid = lax.axis_index("s")

    args = (den_sp, acc_sp, srcv, dstv,
            pend_s, pend_d, pend_v, ov_s, ov_d, ov_v,
            rows, arows, brows, exbuf, dl,
            sem_a, sem_b, sem_c, sid)

    @pl.when(cid == 0)
    def _():
        _sc_process(hd, t1, s1, d1, zacc_h, zden_h, acc1, den1, *args)

    @pl.when(cid == 1)
    def _():
        _sc_process(hg, t2, s2, d2, zacc_h, zden_h, acc2, den2, *args)


def _sc_edge_kernel(hd, hg, t1, t2, s1, d1, s2, d2, zacc_h, zden_h):
    f32 = jnp.float32
    i32 = jnp.int32
    mesh = plsc.VectorSubcoreMesh(core_axis_name="c", subcore_axis_name="s")
    return pl.kernel(
        _sc_body,
        out_type=(
            jax.ShapeDtypeStruct((NOUT, HID), f32),
            jax.ShapeDtypeStruct((NOUT, HEADS), f32),
            jax.ShapeDtypeStruct((NOUT, HID), f32),
            jax.ShapeDtypeStruct((NOUT, HEADS), f32),
        ),
        mesh=mesh,
        compiler_params=pltpu.CompilerParams(use_tc_tiling_on_sc=False,
                                             needs_layout_passes=False),
        scratch_types=[
            pltpu.VMEM_SHARED((NOUT, HEADS), f32),
            pltpu.VMEM_SHARED((R, HID), f32),
            pltpu.VMEM((K,), i32),
            pltpu.VMEM((K,), i32),
            pltpu.VMEM((CEMIT,), i32),
            pltpu.VMEM((CEMIT,), i32),
            pltpu.VMEM((CEMIT,), f32),
            pltpu.VMEM((16,), i32),
            pltpu.VMEM((16,), i32),
            pltpu.VMEM((16,), f32),
            pltpu.VMEM((CEMIT, HID), f32),
            pltpu.VMEM((CEMIT, 2 * HEADS), f32),
            pltpu.VMEM((CEMIT, 2 * HEADS), f32),
            pltpu.VMEM((CEMIT, HEADS), f32),
            pltpu.VMEM((CEMIT,), i32),
            pltpu.SemaphoreType.DMA,
            pltpu.SemaphoreType.DMA,
            pltpu.SemaphoreType.DMA,
        ],
    )(hd, hg, t1, t2, s1, d1, s2, d2, zacc_h, zden_h)


BM = 400
GRID = N // BM


def _proj_body(xg, xd, wg, bg, wd, bd, m1, m2, m34,
               hg, hd, t1, t2):
    g = jnp.dot(xg[...], wg[...], preferred_element_type=jnp.float32) + bg[...]
    d = jnp.dot(xd[...], wd[...], preferred_element_type=jnp.float32) + bd[...]
    hg[...] = g
    hd[...] = d
    t1[...] = (jnp.dot(d, m1[...], preferred_element_type=jnp.float32)
               + jnp.dot(g, m2[...], preferred_element_type=jnp.float32))
    t2[...] = jnp.dot(g, m34[...], preferred_element_type=jnp.float32)


def _proj(xg, xd, wg, bg, wd, bd, m1, m2, m34):
    f32 = jnp.float32
    row = lambda i: (i, 0)
    whole = lambda i: (0, 0)
    return pl.pallas_call(
        _proj_body,
        grid=(GRID,),
        in_specs=[
            pl.BlockSpec((BM, D_IN), row),
            pl.BlockSpec((BM, D_IN), row),
            pl.BlockSpec((D_IN, HID), whole),
            pl.BlockSpec((1, HID), whole),
            pl.BlockSpec((D_IN, HID), whole),
            pl.BlockSpec((1, HID), whole),
            pl.BlockSpec((HID, 2 * HEADS), whole),
            pl.BlockSpec((HID, 2 * HEADS), whole),
            pl.BlockSpec((HID, 2 * HEADS), whole),
        ],
        out_specs=[
            pl.BlockSpec((BM, HID), row),
            pl.BlockSpec((BM, HID), row),
            pl.BlockSpec((BM, 2 * HEADS), row),
            pl.BlockSpec((BM, 2 * HEADS), row),
        ],
        out_shape=[
            jax.ShapeDtypeStruct((N, HID), f32),
            jax.ShapeDtypeStruct((N, HID), f32),
            jax.ShapeDtypeStruct((N, 2 * HEADS), f32),
            jax.ShapeDtypeStruct((N, 2 * HEADS), f32),
        ],
    )(xg, xd, wg, bg, wd, bd, m1, m2, m34)


def _normalize(acc, den, expand):
    r = 1.0 / (den[...] + 1e-16)
    rx = jnp.dot(r, expand[...], preferred_element_type=jnp.float32)
    return jnp.maximum(acc[...] * rx, 0.0)


def _c1_body(acc1, den1, acc2, den2, kw, kb, expand, ks1, ks2):
    i = pl.program_id(0)

    @pl.when(i == 0)
    def _():
        ks1[...] = jnp.zeros_like(ks1)
        ks2[...] = jnp.zeros_like(ks2)

    o1 = _normalize(acc1, den1, expand)
    o2 = _normalize(acc2, den2, expand)
    k1 = jnp.tanh(jnp.dot(o1, kw[...], preferred_element_type=jnp.float32)
                  + kb[...])
    k2 = jnp.tanh(jnp.dot(o2, kw[...], preferred_element_type=jnp.float32)
                  + kb[...])
    ks1[...] += jnp.sum(k1, axis=0, keepdims=True)
    ks2[...] += jnp.sum(k2, axis=0, keepdims=True)


def _c1(acc1, den1, acc2, den2, kw, kb, expand):
    f32 = jnp.float32
    row = lambda i: (i, 0)
    whole = lambda i: (0, 0)
    return pl.pallas_call(
        _c1_body,
        grid=(GRID,),
        in_specs=[
            pl.BlockSpec((BM, HID), row),
            pl.BlockSpec((BM, HEADS), row),
            pl.BlockSpec((BM, HID), row),
            pl.BlockSpec((BM, HEADS), row),
            pl.BlockSpec((HID, HID), whole),
            pl.BlockSpec((1, HID), whole),
            pl.BlockSpec((HEADS, HID), whole),
        ],
        out_specs=[
            pl.BlockSpec((1, HID), whole),
            pl.BlockSpec((1, HID), whole),
        ],
        out_shape=[
            jax.ShapeDtypeStruct((1, HID), f32),
            jax.ShapeDtypeStruct((1, HID), f32),
        ],
    )(acc1, den1, acc2, den2, kw, kb, expand)


def _c2_body(acc1, den1, acc2, den2, attn, lw, lb, expand, out):
    o1 = _normalize(acc1, den1, expand)
    o2 = _normalize(acc2, den2, expand)
    a1 = attn[0, 0]
    a2 = attn[0, 1]
    o = a1 * o1 + a2 * o2
    out[...] = jnp.dot(o, lw[...], preferred_element_type=jnp.float32) + lb[...]


def _c2(acc1, den1, acc2, den2, attn, lw, lb, expand):
    f32 = jnp.float32
    row = lambda i: (i, 0)
    whole = lambda i: (0, 0)
    return pl.pallas_call(
        _c2_body,
        grid=(GRID,),
        in_specs=[
            pl.BlockSpec((BM, HID), row),
            pl.BlockSpec((BM, HEADS), row),
            pl.BlockSpec((BM, HID), row),
            pl.BlockSpec((BM, HEADS), row),
            pl.BlockSpec(memory_space=pltpu.SMEM),
            pl.BlockSpec((HID, OUT), whole),
            pl.BlockSpec((1, OUT), whole),
            pl.BlockSpec((HEADS, HID), whole),
        ],
        out_specs=pl.BlockSpec((BM, OUT), row),
        out_shape=jax.ShapeDtypeStruct((N, OUT), f32),
    )(acc1, den1, acc2, den2, attn, lw, lb, expand)


def _head_mat(a):
    return (a[:, :, None] * jnp.eye(HEADS, dtype=a.dtype)[:, None, :]).reshape(
        HID, HEADS)


def _pad_edges(e):
    return jnp.pad(e, (0, E_PAD - E))


def kernel(x_gene, x_disease, edge_index_g2d, edge_index_d2g, edge_index_g2g,
           proj_gene_w, proj_gene_b, proj_dis_w, proj_dis_b,
           a_src_g2d, a_dst_g2d, a_src_d2g, a_dst_d2g, a_src_g2g, a_dst_g2g,
           k_w, k_b, q, lin_w, lin_b):
    zpad = jnp.zeros((HID, HEADS), jnp.float32)
    m1 = jnp.concatenate([_head_mat(a_src_d2g), zpad], axis=1)
    m2 = jnp.concatenate([zpad, _head_mat(a_dst_d2g)], axis=1)
    m34 = jnp.concatenate([_head_mat(a_src_g2g), _head_mat(a_dst_g2g)],
                          axis=1)
    hg, hd, t1, t2 = _proj(
        x_gene, x_disease, proj_gene_w, proj_gene_b.reshape(1, HID),
        proj_dis_w, proj_dis_b.reshape(1, HID), m1, m2, m34)
    expand = (jnp.eye(HEADS, dtype=jnp.float32)[:, :, None]
              * jnp.ones((1, 1, D_HEAD), jnp.float32)).reshape(HEADS, HID)

    s1 = _pad_edges(edge_index_d2g[0])
    d1 = _pad_edges(edge_index_d2g[1])
    s2 = _pad_edges(edge_index_g2g[0])
    d2 = _pad_edges(edge_index_g2g[1])

    zacc_h = jnp.zeros((RPT, HID), jnp.float32)
    zden_h = jnp.zeros((DPT, HEADS), jnp.float32)
    acc1, den1, acc2, den2 = _sc_edge_kernel(
        hd, hg, t1, t2, s1, d1, s2, d2, zacc_h, zden_h)

    ks1, ks2 = _c1(acc1, den1, acc2, den2, k_w, k_b.reshape(1, HID),
                   expand)
    s_1 = jnp.dot(q, ks1[0] / N)
    s_2 = jnp.dot(q, ks2[0] / N)
    attn = jax.nn.softmax(jnp.stack([s_1, s_2])).reshape(1, 2)

    return _c2(acc1, den1, acc2, den2, attn, lin_w,
               lin_b.reshape(1, OUT), expand)


# speedup vs baseline: 15.3119x; 1.0355x over previous
import functools

import jax
import jax.numpy as jnp
from jax import lax
from jax.experimental import pallas as pl
from jax.experimental.pallas import tpu as pltpu
from jax.experimental.pallas import tpu_sc as plsc

N = 50000
D_IN = 128
HID = 128
HEADS = 8
D_HEAD = 16
OUT = 64
E = 200000

K = 256
NCH = (E + K - 1) // K
E_PAD = NCH * K
R = 10240
NPASS = 5
NOUT = R * NPASS
CEMIT = 128
RPT = R // 16
DPT = NOUT // 16


def _iota16():
    return lax.broadcasted_iota(jnp.int32, (16,), 0)


def _lrelu_exp(a):
    x = jnp.where(a >= 0.0, a, 0.2 * a)
    return jnp.exp(x)


def _sc_process(xsrc, atab, src_h, dst_h, zacc_h, zden_h,
                acc_o, den_o,
                den_sp, acc_sp, srcv, dstv,
                pend_s, pend_d, pend_v, ov_s, ov_d, ov_v,
                rows, arows, brows, exbuf, dl,
                sem_a, sem_b, sem_c, sid):
    it16 = _iota16()
    nmy = (NCH - sid + 15) // 16

    pltpu.sync_copy(zden_h, den_sp.at[pl.ds(sid * DPT, DPT)])

    r0 = sid * RPT

    def p2_pass(p, carry):
        base = p * R

        pltpu.sync_copy(zacc_h, acc_sp.at[pl.ds(r0, RPT)])
        plsc.subcore_barrier()

        def emit(fill_after):
            dx = pltpu.async_copy(xsrc.at[pend_s], rows, sem_a)
            da = pltpu.async_copy(atab.at[pend_s], arows, sem_b)
            db = pltpu.async_copy(atab.at[pend_d], brows, sem_c)
            da.wait()
            db.wait()

            def exj(j, c2):
                e16 = j * 16 + it16
                v = pend_v[pl.ds(j * 16, 16)]
                for h in range(HEADS):
                    h16 = jnp.full((16,), h, jnp.int32)
                    a = plsc.load_gather(arows, [e16, h16])
                    b = plsc.load_gather(brows, [e16, h16 + 8])
                    ex = _lrelu_exp(a + b) * v
                    plsc.store_scatter(exbuf, [e16, h16], ex)
                    plsc.store_scatter(brows, [e16, h16], ex)
                return c2

            lax.fori_loop(0, CEMIT // 16, exj, 0)
            pltpu.sync_copy(exbuf, den_sp.at[pend_d], add=True)

            for j in range(CEMIT // 16):
                d16 = pend_d[pl.ds(j * 16, 16)]
                dl[pl.ds(j * 16, 16)] = d16 - base

            dx.wait()

            gdn = lax.GatherDimensionNumbers(
                offset_dims=(), collapsed_slice_dims=(0,),
                start_index_map=(0,))

            def scale(e, c2):
                ex16 = brows[e, pl.ds(0, 16)]
                for h in range(HEADS):
                    sv = lax.gather(
                        ex16, jnp.full((16, 1), h, jnp.int32), gdn,
                        slice_sizes=(1,),
                        mode=lax.GatherScatterMode.PROMISE_IN_BOUNDS)
                    rows[e, pl.ds(h * 16, 16)] = (
                        rows[e, pl.ds(h * 16, 16)] * sv)
                return c2

            lax.fori_loop(0, CEMIT, scale, 0)
            pltpu.sync_copy(rows, acc_sp.at[dl], add=True)
            return fill_after

        def p2_chunk(gl, fill):
            g = sid + gl * 16
            base_e = g * K
            pltpu.sync_copy(src_h.at[pl.ds(base_e, K)], srcv)
            pltpu.sync_copy(dst_h.at[pl.ds(base_e, K)], dstv)

            def p2_vec(i, fill):
                e16 = i * 16 + it16
                eid = base_e + e16
                s16 = srcv[pl.ds(i * 16, 16)]
                d16 = dstv[pl.ds(i * 16, 16)]
                m = (d16 >= base) & (d16 < base + R) & (eid < E)
                cnt = jnp.sum(m.astype(jnp.int32))
                pos = fill + plsc.cumsum(m.astype(jnp.int32)) - 1
                m_lo = m & (pos < CEMIT)
                m_hi = m & (pos >= CEMIT)
                one16 = jnp.full((16,), 1.0, jnp.float32)
                plsc.store_scatter(pend_s, [pos], s16, mask=m_lo)
                plsc.store_scatter(pend_d, [pos], d16, mask=m_lo)
                plsc.store_scatter(pend_v, [pos], one16, mask=m_lo)
                plsc.store_scatter(ov_s, [pos - CEMIT], s16, mask=m_hi)
                plsc.store_scatter(ov_d, [pos - CEMIT], d16, mask=m_hi)
                plsc.store_scatter(ov_v, [pos - CEMIT], one16, mask=m_hi)
                fill = fill + cnt

                def do_emit(f):
                    f = emit(f - CEMIT)
                    pend_s[pl.ds(0, 16)] = ov_s[...]
                    pend_d[pl.ds(0, 16)] = ov_d[...]
                    pend_v[pl.ds(0, 16)] = ov_v[...]
                    return f

                return lax.cond(fill >= CEMIT, do_emit, lambda f: f, fill)

            return lax.fori_loop(0, K // 16, p2_vec, fill)

        fill = lax.fori_loop(0, nmy, p2_chunk, jnp.int32(0))

        dummy_s = sid * 16 + it16
        zero16 = jnp.zeros((16,), jnp.float32)
        for jj in range(CEMIT // 16):
            pos = fill + jj * 16 + it16
            mk = pos < CEMIT
            dummy_d = base + sid * CEMIT + jj * 16 + it16
            plsc.store_scatter(pend_s, [pos], dummy_s, mask=mk)
            plsc.store_scatter(pend_d, [pos], dummy_d, mask=mk)
            plsc.store_scatter(pend_v, [pos], zero16, mask=mk)
        emit(0)

        plsc.subcore_barrier()
        pltpu.sync_copy(acc_sp.at[pl.ds(r0, RPT)],
                        acc_o.at[pl.ds(base + r0, RPT)])
        plsc.subcore_barrier()
        return carry

    lax.fori_loop(0, NPASS, p2_pass, 0)
    pltpu.sync_copy(den_sp.at[pl.ds(sid * DPT, DPT)],
                    den_o.at[pl.ds(sid * DPT, DPT)])


def _sc_body(hd, hg, t1, t2, s1, d1, s2, d2, zacc_h, zden_h,
             acc1, den1, acc2, den2,
             den_sp, acc_sp, srcv, dstv,
             pend_s, pend_d, pend_v, ov_s, ov_d, ov_v,
             rows, arows, brows, exbuf, dl,
             sem_a, sem_b, sem_c):
    cid = lax.axis_index("c")
    sid = lax.axis_index("s")

    args = (den_sp, acc_sp, srcv, dstv,
            pend_s, pend_d, pend_v, ov_s, ov_d, ov_v,
            rows, arows, brows, exbuf, dl,
            sem_a, sem_b, sem_c, sid)

    @pl.when(cid == 0)
    def _():
        _sc_process(hd, t1, s1, d1, zacc_h, zden_h, acc1, den1, *args)

    @pl.when(cid == 1)
    def _():
        _sc_process(hg, t2, s2, d2, zacc_h, zden_h, acc2, den2, *args)


def _sc_edge_kernel(hd, hg, t1, t2, s1, d1, s2, d2, zacc_h, zden_h):
    f32 = jnp.float32
    i32 = jnp.int32
    mesh = plsc.VectorSubcoreMesh(core_axis_name="c", subcore_axis_name="s")
    return pl.kernel(
        _sc_body,
        out_type=(
            jax.ShapeDtypeStruct((NOUT, HID), f32),
            jax.ShapeDtypeStruct((NOUT, HEADS), f32),
            jax.ShapeDtypeStruct((NOUT, HID), f32),
            jax.ShapeDtypeStruct((NOUT, HEADS), f32),
        ),
        mesh=mesh,
        compiler_params=pltpu.CompilerParams(use_tc_tiling_on_sc=False,
                                             needs_layout_passes=False),
        scratch_types=[
            pltpu.VMEM_SHARED((NOUT, HEADS), f32),
            pltpu.VMEM_SHARED((R, HID), f32),
            pltpu.VMEM((K,), i32),
            pltpu.VMEM((K,), i32),
            pltpu.VMEM((CEMIT,), i32),
            pltpu.VMEM((CEMIT,), i32),
            pltpu.VMEM((CEMIT,), f32),
            pltpu.VMEM((16,), i32),
            pltpu.VMEM((16,), i32),
            pltpu.VMEM((16,), f32),
            pltpu.VMEM((CEMIT, HID), f32),
            pltpu.VMEM((CEMIT, 2 * HEADS), f32),
            pltpu.VMEM((CEMIT, 2 * HEADS), f32),
            pltpu.VMEM((CEMIT, HEADS), f32),
            pltpu.VMEM((CEMIT,), i32),
            pltpu.SemaphoreType.DMA,
            pltpu.SemaphoreType.DMA,
            pltpu.SemaphoreType.DMA,
        ],
    )(hd, hg, t1, t2, s1, d1, s2, d2, zacc_h, zden_h)


BM = 400
GRID = N // BM


def _proj_body(xg, xd, wg, bg, wd, bd, m1, m2, m34,
               hg, hd, t1, t2):
    g = jnp.dot(xg[...], wg[...], preferred_element_type=jnp.float32) + bg[...]
    d = jnp.dot(xd[...], wd[...], preferred_element_type=jnp.float32) + bd[...]
    hg[...] = g
    hd[...] = d
    t1[...] = (jnp.dot(d, m1[...], preferred_element_type=jnp.float32)
               + jnp.dot(g, m2[...], preferred_element_type=jnp.float32))
    t2[...] = jnp.dot(g, m34[...], preferred_element_type=jnp.float32)


def _proj(xg, xd, wg, bg, wd, bd, m1, m2, m34):
    f32 = jnp.float32
    row = lambda i: (i, 0)
    whole = lambda i: (0, 0)
    return pl.pallas_call(
        _proj_body,
        grid=(GRID,),
        in_specs=[
            pl.BlockSpec((BM, D_IN), row),
            pl.BlockSpec((BM, D_IN), row),
            pl.BlockSpec((D_IN, HID), whole),
            pl.BlockSpec((1, HID), whole),
            pl.BlockSpec((D_IN, HID), whole),
            pl.BlockSpec((1, HID), whole),
            pl.BlockSpec((HID, 2 * HEADS), whole),
            pl.BlockSpec((HID, 2 * HEADS), whole),
            pl.BlockSpec((HID, 2 * HEADS), whole),
        ],
        out_specs=[
            pl.BlockSpec((BM, HID), row),
            pl.BlockSpec((BM, HID), row),
            pl.BlockSpec((BM, 2 * HEADS), row),
            pl.BlockSpec((BM, 2 * HEADS), row),
        ],
        out_shape=[
            jax.ShapeDtypeStruct((N, HID), f32),
            jax.ShapeDtypeStruct((N, HID), f32),
            jax.ShapeDtypeStruct((N, 2 * HEADS), f32),
            jax.ShapeDtypeStruct((N, 2 * HEADS), f32),
        ],
    )(xg, xd, wg, bg, wd, bd, m1, m2, m34)


def _normalize(acc, den, expand):
    r = 1.0 / (den[...] + 1e-16)
    rx = jnp.dot(r, expand[...], preferred_element_type=jnp.float32)
    return jnp.maximum(acc[...] * rx, 0.0)


def _c1_body(acc1, den1, acc2, den2, kw, kb, expand, ks1, ks2):
    i = pl.program_id(0)

    @pl.when(i == 0)
    def _():
        ks1[...] = jnp.zeros_like(ks1)
        ks2[...] = jnp.zeros_like(ks2)

    o1 = _normalize(acc1, den1, expand)
    o2 = _normalize(acc2, den2, expand)
    k1 = jnp.tanh(jnp.dot(o1, kw[...], preferred_element_type=jnp.float32)
                  + kb[...])
    k2 = jnp.tanh(jnp.dot(o2, kw[...], preferred_element_type=jnp.float32)
                  + kb[...])
    ks1[...] += jnp.sum(k1, axis=0, keepdims=True)
    ks2[...] += jnp.sum(k2, axis=0, keepdims=True)


def _c1(acc1, den1, acc2, den2, kw, kb, expand):
    f32 = jnp.float32
    row = lambda i: (i, 0)
    whole = lambda i: (0, 0)
    return pl.pallas_call(
        _c1_body,
        grid=(GRID,),
        in_specs=[
            pl.BlockSpec((BM, HID), row),
            pl.BlockSpec((BM, HEADS), row),
            pl.BlockSpec((BM, HID), row),
            pl.BlockSpec((BM, HEADS), row),
            pl.BlockSpec((HID, HID), whole),
            pl.BlockSpec((1, HID), whole),
            pl.BlockSpec((HEADS, HID), whole),
        ],
        out_specs=[
            pl.BlockSpec((1, HID), whole),
            pl.BlockSpec((1, HID), whole),
        ],
        out_shape=[
            jax.ShapeDtypeStruct((1, HID), f32),
            jax.ShapeDtypeStruct((1, HID), f32),
        ],
    )(acc1, den1, acc2, den2, kw, kb, expand)


def _c2_body(acc1, den1, acc2, den2, attn, lw, lb, expand, out):
    o1 = _normalize(acc1, den1, expand)
    o2 = _normalize(acc2, den2, expand)
    a1 = attn[0, 0]
    a2 = attn[0, 1]
    o = a1 * o1 + a2 * o2
    out[...] = jnp.dot(o, lw[...], preferred_element_type=jnp.float32) + lb[...]


def _c2(acc1, den1, acc2, den2, attn, lw, lb, expand):
    f32 = jnp.float32
    row = lambda i: (i, 0)
    whole = lambda i: (0, 0)
    return pl.pallas_call(
        _c2_body,
        grid=(GRID,),
        in_specs=[
            pl.BlockSpec((BM, HID), row),
            pl.BlockSpec((BM, HEADS), row),
            pl.BlockSpec((BM, HID), row),
            pl.BlockSpec((BM, HEADS), row),
            pl.BlockSpec(memory_space=pltpu.SMEM),
            pl.BlockSpec((HID, OUT), whole),
            pl.BlockSpec((1, OUT), whole),
            pl.BlockSpec((HEADS, HID), whole),
        ],
        out_specs=pl.BlockSpec((BM, OUT), row),
        out_shape=jax.ShapeDtypeStruct((N, OUT), f32),
    )(acc1, den1, acc2, den2, attn, lw, lb, expand)


def _head_mat(a):
    return (a[:, :, None] * jnp.eye(HEADS, dtype=a.dtype)[:, None, :]).reshape(
        HID, HEADS)


def _pad_edges(e):
    return jnp.pad(e, (0, E_PAD - E))


def kernel(x_gene, x_disease, edge_index_g2d, edge_index_d2g, edge_index_g2g,
           proj_gene_w, proj_gene_b, proj_dis_w, proj_dis_b,
           a_src_g2d, a_dst_g2d, a_src_d2g, a_dst_d2g, a_src_g2g, a_dst_g2g,
           k_w, k_b, q, lin_w, lin_b):
    zpad = jnp.zeros((HID, HEADS), jnp.float32)
    m1 = jnp.concatenate([_head_mat(a_src_d2g), zpad], axis=1)
    m2 = jnp.concatenate([zpad, _head_mat(a_dst_d2g)], axis=1)
    m34 = jnp.concatenate([_head_mat(a_src_g2g), _head_mat(a_dst_g2g)],
                          axis=1)
    hg, hd, t1, t2 = _proj(
        x_gene, x_disease, proj_gene_w, proj_gene_b.reshape(1, HID),
        proj_dis_w, proj_dis_b.reshape(1, HID), m1, m2, m34)
    expand = (jnp.eye(HEADS, dtype=jnp.float32)[:, :, None]
              * jnp.ones((1, 1, D_HEAD), jnp.float32)).reshape(HEADS, HID)

    s1 = _pad_edges(edge_index_d2g[0])
    d1 = _pad_edges(edge_index_d2g[1])
    s2 = _pad_edges(edge_index_g2g[0])
    d2 = _pad_edges(edge_index_g2g[1])

    zacc_h = jnp.zeros((RPT, HID), jnp.float32)
    zden_h = jnp.zeros((DPT, HEADS), jnp.float32)
    acc1, den1, acc2, den2 = _sc_edge_kernel(
        hd, hg, t1, t2, s1, d1, s2, d2, zacc_h, zden_h)

    ks1, ks2 = _c1(acc1, den1, acc2, den2, k_w, k_b.reshape(1, HID),
                   expand)
    s_1 = jnp.dot(q, ks1[0] / N)
    s_2 = jnp.dot(q, ks2[0] / N)
    attn = jax.nn.softmax(jnp.stack([s_1, s_2])).reshape(1, 2)

    return _c2(acc1, den1, acc2, den2, attn, lin_w,
               lin_b.reshape(1, OUT), expand)


# speedup vs baseline: 15.3859x; 1.0048x over previous
import functools

import jax
import jax.numpy as jnp
from jax import lax
from jax.experimental import pallas as pl
from jax.experimental.pallas import tpu as pltpu
from jax.experimental.pallas import tpu_sc as plsc

N = 50000
D_IN = 128
HID = 128
HEADS = 8
D_HEAD = 16
OUT = 64
E = 200000

K = 256
NCH = (E + K - 1) // K
E_PAD = NCH * K
R = 10240
NPASS = 5
NOUT = R * NPASS
CEMIT = 128
RPT = R // 16
DPT = NOUT // 16


def _iota16():
    return lax.broadcasted_iota(jnp.int32, (16,), 0)


def _lrelu_exp(a):
    x = jnp.where(a >= 0.0, a, 0.2 * a)
    return jnp.exp(x)


def _sc_process(xsrc, atab, src_h, dst_h, zacc_h, zden_h,
                acc_o, den_o,
                den_sp, acc_sp, srcv, dstv,
                pend_s, pend_d, pend_v, ov_s, ov_d, ov_v,
                rows, arows, brows, exbuf, dl,
                sem_a, sem_b, sem_c, sid):
    it16 = _iota16()
    nmy = (NCH - sid + 15) // 16

    pltpu.sync_copy(zden_h, den_sp.at[pl.ds(sid * DPT, DPT)])

    r0 = sid * RPT

    def p2_pass(p, carry):
        base = p * R

        pltpu.sync_copy(zacc_h, acc_sp.at[pl.ds(r0, RPT)])
        plsc.subcore_barrier()

        def emit(fill_after):
            dx = pltpu.async_copy(xsrc.at[pend_s], rows, sem_a)
            da = pltpu.async_copy(atab.at[pend_s], arows, sem_b)
            db = pltpu.async_copy(atab.at[pend_d], brows, sem_c)
            da.wait()
            db.wait()

            def exj(j, c2):
                e16 = j * 16 + it16
                v = pend_v[pl.ds(j * 16, 16)]
                for h in range(HEADS):
                    h16 = jnp.full((16,), h, jnp.int32)
                    a = plsc.load_gather(arows, [e16, h16])
                    b = plsc.load_gather(brows, [e16, h16 + 8])
                    ex = _lrelu_exp(a + b) * v
                    plsc.store_scatter(exbuf, [e16, h16], ex)
                    plsc.store_scatter(brows, [e16, h16], ex)
                return c2

            lax.fori_loop(0, CEMIT // 16, exj, 0)
            pltpu.sync_copy(exbuf, den_sp.at[pend_d], add=True)

            for j in range(CEMIT // 16):
                d16 = pend_d[pl.ds(j * 16, 16)]
                dl[pl.ds(j * 16, 16)] = d16 - base

            dx.wait()

            gdn = lax.GatherDimensionNumbers(
                offset_dims=(), collapsed_slice_dims=(0,),
                start_index_map=(0,))

            def scale(e, c2):
                ex16 = brows[e, pl.ds(0, 16)]
                for h in range(HEADS):
                    sv = lax.gather(
                        ex16, jnp.full((16, 1), h, jnp.int32), gdn,
                        slice_sizes=(1,),
                        mode=lax.GatherScatterMode.PROMISE_IN_BOUNDS)
                    rows[e, pl.ds(h * 16, 16)] = (
                        rows[e, pl.ds(h * 16, 16)] * sv)
                return c2

            lax.fori_loop(0, CEMIT, scale, 0)
            pltpu.sync_copy(rows, acc_sp.at[dl], add=True)
            return fill_after

        def p2_chunk(gl, fill):
            g = sid + gl * 16
            base_e = g * K
            pltpu.sync_copy(src_h.at[pl.ds(base_e, K)], srcv)
            pltpu.sync_copy(dst_h.at[pl.ds(base_e, K)], dstv)

            def p2_vec(i, fill):
                e16 = i * 16 + it16
                eid = base_e + e16
                s16 = srcv[pl.ds(i * 16, 16)]
                d16 = dstv[pl.ds(i * 16, 16)]
                m = (d16 >= base) & (d16 < base + R) & (eid < E)
                cnt = jnp.sum(m.astype(jnp.int32))
                pos = fill + plsc.cumsum(m.astype(jnp.int32)) - 1
                m_lo = m & (pos < CEMIT)
                m_hi = m & (pos >= CEMIT)
                one16 = jnp.full((16,), 1.0, jnp.float32)
                plsc.store_scatter(pend_s, [pos], s16, mask=m_lo)
                plsc.store_scatter(pend_d, [pos], d16, mask=m_lo)
                plsc.store_scatter(pend_v, [pos], one16, mask=m_lo)
                plsc.store_scatter(ov_s, [pos - CEMIT], s16, mask=m_hi)
                plsc.store_scatter(ov_d, [pos - CEMIT], d16, mask=m_hi)
                plsc.store_scatter(ov_v, [pos - CEMIT], one16, mask=m_hi)
                fill = fill + cnt

                def do_emit(f):
                    f = emit(f - CEMIT)
                    pend_s[pl.ds(0, 16)] = ov_s[...]
                    pend_d[pl.ds(0, 16)] = ov_d[...]
                    pend_v[pl.ds(0, 16)] = ov_v[...]
                    return f

                return lax.cond(fill >= CEMIT, do_emit, lambda f: f, fill)

            return lax.fori_loop(0, K // 16, p2_vec, fill)

        fill = lax.fori_loop(0, nmy, p2_chunk, jnp.int32(0))

        dummy_s = sid * 16 + it16
        zero16 = jnp.zeros((16,), jnp.float32)
        for jj in range(CEMIT // 16):
            pos = fill + jj * 16 + it16
            mk = pos < CEMIT
            dummy_d = base + sid * CEMIT + jj * 16 + it16
            plsc.store_scatter(pend_s, [pos], dummy_s, mask=mk)
            plsc.store_scatter(pend_d, [pos], dummy_d, mask=mk)
            plsc.store_scatter(pend_v, [pos], zero16, mask=mk)
        emit(0)

        plsc.subcore_barrier()
        pltpu.sync_copy(acc_sp.at[pl.ds(r0, RPT)],
                        acc_o.at[pl.ds(base + r0, RPT)])
        plsc.subcore_barrier()
        return carry

    lax.fori_loop(0, NPASS, p2_pass, 0)
    pltpu.sync_copy(den_sp.at[pl.ds(sid * DPT, DPT)],
                    den_o.at[pl.ds(sid * DPT, DPT)])


def _sc_body(hd, hg, t1, t2, s1, d1, s2, d2, zacc_h, zden_h,
             acc1, den1, acc2, den2,
             den_sp, acc_sp, srcv, dstv,
             pend_s, pend_d, pend_v, ov_s, ov_d, ov_v,
             rows, arows, brows, exbuf, dl,
             sem_a, sem_b, sem_c):
    cid = lax.axis_index("c")
    sid = lax.axis_index("s")

    args = (den_sp, acc_sp, srcv, dstv,
            pend_s, pend_d, pend_v, ov_s, ov_d, ov_v,
            rows, arows, brows, exbuf, dl,
            sem_a, sem_b, sem_c, sid)

    @pl.when(cid == 0)
    def _():
        _sc_process(hd, t1, s1, d1, zacc_h, zden_h, acc1, den1, *args)

    @pl.when(cid == 1)
    def _():
        _sc_process(hg, t2, s2, d2, zacc_h, zden_h, acc2, den2, *args)


def _sc_edge_kernel(hd, hg, t1, t2, s1, d1, s2, d2, zacc_h, zden_h):
    f32 = jnp.float32
    i32 = jnp.int32
    mesh = plsc.VectorSubcoreMesh(core_axis_name="c", subcore_axis_name="s")
    return pl.kernel(
        _sc_body,
        out_type=(
            jax.ShapeDtypeStruct((NOUT, HID), f32),
            jax.ShapeDtypeStruct((NOUT, HEADS), f32),
            jax.ShapeDtypeStruct((NOUT, HID), f32),
            jax.ShapeDtypeStruct((NOUT, HEADS), f32),
        ),
        mesh=mesh,
        compiler_params=pltpu.CompilerParams(use_tc_tiling_on_sc=False,
                                             needs_layout_passes=False),
        scratch_types=[
            pltpu.VMEM_SHARED((NOUT, HEADS), f32),
            pltpu.VMEM_SHARED((R, HID), f32),
            pltpu.VMEM((K,), i32),
            pltpu.VMEM((K,), i32),
            pltpu.VMEM((CEMIT,), i32),
            pltpu.VMEM((CEMIT,), i32),
            pltpu.VMEM((CEMIT,), f32),
            pltpu.VMEM((16,), i32),
            pltpu.VMEM((16,), i32),
            pltpu.VMEM((16,), f32),
            pltpu.VMEM((CEMIT, HID), f32),
            pltpu.VMEM((CEMIT, 2 * HEADS), f32),
            pltpu.VMEM((CEMIT, 2 * HEADS), f32),
            pltpu.VMEM((CEMIT, HEADS), f32),
            pltpu.VMEM((CEMIT,), i32),
            pltpu.SemaphoreType.DMA,
            pltpu.SemaphoreType.DMA,
            pltpu.SemaphoreType.DMA,
        ],
    )(hd, hg, t1, t2, s1, d1, s2, d2, zacc_h, zden_h)


BM = 400
GRID = N // BM


def _head_block_diag(vrow):
    rr = lax.broadcasted_iota(jnp.int32, (HID, HEADS), 0)
    cc = lax.broadcasted_iota(jnp.int32, (HID, HEADS), 1)
    return jnp.where(rr // D_HEAD == cc, vrow[0][:, None], 0.0)


def _proj_body(xg, xd, wg, bg, wd, bd, as1, ad1, as2, ad2,
               hg, hd, t1, t2):
    g = jnp.dot(xg[...], wg[...], preferred_element_type=jnp.float32) + bg[...]
    d = jnp.dot(xd[...], wd[...], preferred_element_type=jnp.float32) + bd[...]
    hg[...] = g
    hd[...] = d
    t1[...] = jnp.concatenate(
        [jnp.dot(d, _head_block_diag(as1), preferred_element_type=jnp.float32),
         jnp.dot(g, _head_block_diag(ad1), preferred_element_type=jnp.float32)],
        axis=1)
    t2[...] = jnp.concatenate(
        [jnp.dot(g, _head_block_diag(as2), preferred_element_type=jnp.float32),
         jnp.dot(g, _head_block_diag(ad2), preferred_element_type=jnp.float32)],
        axis=1)


def _proj(xg, xd, wg, bg, wd, bd, as1, ad1, as2, ad2):
    f32 = jnp.float32
    row = lambda i: (i, 0)
    whole = lambda i: (0, 0)
    return pl.pallas_call(
        _proj_body,
        grid=(GRID,),
        in_specs=[
            pl.BlockSpec((BM, D_IN), row),
            pl.BlockSpec((BM, D_IN), row),
            pl.BlockSpec((D_IN, HID), whole),
            pl.BlockSpec((1, HID), whole),
            pl.BlockSpec((D_IN, HID), whole),
            pl.BlockSpec((1, HID), whole),
            pl.BlockSpec((1, HID), whole),
            pl.BlockSpec((1, HID), whole),
            pl.BlockSpec((1, HID), whole),
            pl.BlockSpec((1, HID), whole),
        ],
        out_specs=[
            pl.BlockSpec((BM, HID), row),
            pl.BlockSpec((BM, HID), row),
            pl.BlockSpec((BM, 2 * HEADS), row),
            pl.BlockSpec((BM, 2 * HEADS), row),
        ],
        out_shape=[
            jax.ShapeDtypeStruct((N, HID), f32),
            jax.ShapeDtypeStruct((N, HID), f32),
            jax.ShapeDtypeStruct((N, 2 * HEADS), f32),
            jax.ShapeDtypeStruct((N, 2 * HEADS), f32),
        ],
    )(xg, xd, wg, bg, wd, bd, as1, ad1, as2, ad2)


def _normalize(acc, den, expand):
    r = 1.0 / (den[...] + 1e-16)
    rx = jnp.dot(r, expand[...], preferred_element_type=jnp.float32)
    return jnp.maximum(acc[...] * rx, 0.0)


def _c1_body(acc1, den1, acc2, den2, kw, kb, expand, ks1, ks2):
    i = pl.program_id(0)

    @pl.when(i == 0)
    def _():
        ks1[...] = jnp.zeros_like(ks1)
        ks2[...] = jnp.zeros_like(ks2)

    o1 = _normalize(acc1, den1, expand)
    o2 = _normalize(acc2, den2, expand)
    k1 = jnp.tanh(jnp.dot(o1, kw[...], preferred_element_type=jnp.float32)
                  + kb[...])
    k2 = jnp.tanh(jnp.dot(o2, kw[...], preferred_element_type=jnp.float32)
                  + kb[...])
    ks1[...] += jnp.sum(k1, axis=0, keepdims=True)
    ks2[...] += jnp.sum(k2, axis=0, keepdims=True)


def _c1(acc1, den1, acc2, den2, kw, kb, expand):
    f32 = jnp.float32
    row = lambda i: (i, 0)
    whole = lambda i: (0, 0)
    return pl.pallas_call(
        _c1_body,
        grid=(GRID,),
        in_specs=[
            pl.BlockSpec((BM, HID), row),
            pl.BlockSpec((BM, HEADS), row),
            pl.BlockSpec((BM, HID), row),
            pl.BlockSpec((BM, HEADS), row),
            pl.BlockSpec((HID, HID), whole),
            pl.BlockSpec((1, HID), whole),
            pl.BlockSpec((HEADS, HID), whole),
        ],
        out_specs=[
            pl.BlockSpec((1, HID), whole),
            pl.BlockSpec((1, HID), whole),
        ],
        out_shape=[
            jax.ShapeDtypeStruct((1, HID), f32),
            jax.ShapeDtypeStruct((1, HID), f32),
        ],
    )(acc1, den1, acc2, den2, kw, kb, expand)


def _c2_body(acc1, den1, acc2, den2, ks1, ks2, qr, lw, lb, expand, out):
    o1 = _normalize(acc1, den1, expand)
    o2 = _normalize(acc2, den2, expand)
    s1 = jnp.sum(ks1[...] * qr[...]) * (1.0 / N)
    s2 = jnp.sum(ks2[...] * qr[...]) * (1.0 / N)
    m = jnp.maximum(s1, s2)
    e1 = jnp.exp(s1 - m)
    e2 = jnp.exp(s2 - m)
    r = 1.0 / (e1 + e2)
    o = (e1 * r) * o1 + (e2 * r) * o2
    out[...] = jnp.dot(o, lw[...], preferred_element_type=jnp.float32) + lb[...]


def _c2(acc1, den1, acc2, den2, ks1, ks2, qr, lw, lb, expand):
    f32 = jnp.float32
    row = lambda i: (i, 0)
    whole = lambda i: (0, 0)
    return pl.pallas_call(
        _c2_body,
        grid=(GRID,),
        in_specs=[
            pl.BlockSpec((BM, HID), row),
            pl.BlockSpec((BM, HEADS), row),
            pl.BlockSpec((BM, HID), row),
            pl.BlockSpec((BM, HEADS), row),
            pl.BlockSpec((1, HID), whole),
            pl.BlockSpec((1, HID), whole),
            pl.BlockSpec((1, HID), whole),
            pl.BlockSpec((HID, OUT), whole),
            pl.BlockSpec((1, OUT), whole),
            pl.BlockSpec((HEADS, HID), whole),
        ],
        out_specs=pl.BlockSpec((BM, OUT), row),
        out_shape=jax.ShapeDtypeStruct((N, OUT), f32),
    )(acc1, den1, acc2, den2, ks1, ks2, qr, lw, lb, expand)


def _pad_edges(e):
    return jnp.pad(e, (0, E_PAD - E))


def kernel(x_gene, x_disease, edge_index_g2d, edge_index_d2g, edge_index_g2g,
           proj_gene_w, proj_gene_b, proj_dis_w, proj_dis_b,
           a_src_g2d, a_dst_g2d, a_src_d2g, a_dst_d2g, a_src_g2g, a_dst_g2g,
           k_w, k_b, q, lin_w, lin_b):
    hg, hd, t1, t2 = _proj(
        x_gene, x_disease, proj_gene_w, proj_gene_b.reshape(1, HID),
        proj_dis_w, proj_dis_b.reshape(1, HID),
        a_src_d2g.reshape(1, HID), a_dst_d2g.reshape(1, HID),
        a_src_g2g.reshape(1, HID), a_dst_g2g.reshape(1, HID))
    expand = (jnp.eye(HEADS, dtype=jnp.float32)[:, :, None]
              * jnp.ones((1, 1, D_HEAD), jnp.float32)).reshape(HEADS, HID)

    s1 = _pad_edges(edge_index_d2g[0])
    d1 = _pad_edges(edge_index_d2g[1])
    s2 = _pad_edges(edge_index_g2g[0])
    d2 = _pad_edges(edge_index_g2g[1])

    zacc_h = jnp.zeros((RPT, HID), jnp.float32)
    zden_h = jnp.zeros((DPT, HEADS), jnp.float32)
    acc1, den1, acc2, den2 = _sc_edge_kernel(
        hd, hg, t1, t2, s1, d1, s2, d2, zacc_h, zden_h)

    ks1, ks2 = _c1(acc1, den1, acc2, den2, k_w, k_b.reshape(1, HID),
                   expand)

    return _c2(acc1, den1, acc2, den2, ks1, ks2, q.reshape(1, HID),
               lin_w, lin_b.reshape(1, OUT), expand)
